# Initial kernel scaffold; baseline (speedup 1.0000x reference)
import jax
import jax.numpy as jnp
from jax.experimental import pallas as pl


def kernel(x, edge_index, W0, b0, W1, b1, W2, b2, p, Wlin, blin):
    raise NotImplementedError("write your pallas kernel here")


# speedup vs baseline: 6.9357x; 6.9357x over previous
import functools

import jax
import jax.numpy as jnp
from jax import lax
from jax.experimental import pallas as pl
from jax.experimental.pallas import tpu as pltpu
from jax.experimental.pallas import tpu_sc as plsc

_N = 10000
_E = 320000
_F = 128
_NCLS = 40
_NC = 2
_NS = 16
_NW = _NC * _NS
_C = 80
_EPW = _E // _NW
_CPW = _EPW // _C
_NZ = 10
_RPS = _N // _NZ


def _sc_agg_body(h_hbm, src_hbm, dst_hbm, zero_hbm, out_hbm,
                 src_v, dst_v, rows_v, agg_sh, sem):
    cid = lax.axis_index("c")
    sid = lax.axis_index("s")
    w = sid * _NC + cid

    pltpu.sync_copy(src_hbm.at[w], src_v)
    pltpu.sync_copy(dst_hbm.at[w], dst_v)
    @pl.when(sid < _NZ)
    def _zero():
        pltpu.sync_copy(zero_hbm.at[pl.ds(sid * _RPS, _RPS)],
                        agg_sh.at[pl.ds(sid * _RPS, _RPS)])
    plsc.subcore_barrier()

    def chunk(j, carry):
        pltpu.async_copy(h_hbm.at[src_v.at[j]], rows_v, sem).wait()
        pltpu.sync_copy(rows_v, agg_sh.at[dst_v.at[j]], add=True)
        return carry

    lax.fori_loop(0, _CPW, chunk, 0)

    plsc.subcore_barrier()

    @pl.when(sid < _NZ)
    def _writeout():
        pltpu.sync_copy(agg_sh.at[pl.ds(sid * _RPS, _RPS)],
                        out_hbm.at[pl.ds(cid * _N + sid * _RPS, _RPS)])


_sc_aggregate = pl.kernel(
    _sc_agg_body,
    out_type=jax.ShapeDtypeStruct((2 * _N, _F), jnp.float32),
    mesh=plsc.VectorSubcoreMesh(core_axis_name="c", subcore_axis_name="s"),
    scratch_types=[
        pltpu.VMEM((_CPW, _C), jnp.int32),
        pltpu.VMEM((_CPW, _C), jnp.int32),
        pltpu.VMEM((_C, _F), jnp.float32),
        pltpu.VMEM_SHARED((_N, _F), jnp.float32),
        pltpu.SemaphoreType.DMA,
    ],
)


def _dense_body(x_ref, a0_ref, a1_ref, w_ref, b_ref, p_ref, o_ref):
    s = x_ref[...] + a0_ref[...] + a1_ref[...]
    y = jnp.dot(s, w_ref[...], preferred_element_type=jnp.float32) + b_ref[...]
    y = jnp.maximum(y, 0.0)
    o_ref[...] = y * jnp.clip(p_ref[...], 0.0, 1.0)


def _dense_final_body(x_ref, a0_ref, a1_ref, w_ref, b_ref, p_ref,
                      wl_ref, bl_ref, o_ref):
    s = x_ref[...] + a0_ref[...] + a1_ref[...]
    y = jnp.dot(s, w_ref[...], preferred_element_type=jnp.float32) + b_ref[...]
    y = jnp.maximum(y, 0.0)
    y = y * jnp.clip(p_ref[...], 0.0, 1.0)
    o_ref[...] = jnp.dot(y, wl_ref[...], preferred_element_type=jnp.float32) + bl_ref[...]


_BR = 1000


def _row_spec():
    return pl.BlockSpec((_BR, _F), lambda i: (i, 0))


def _full_spec():
    return pl.BlockSpec((_F, _F), lambda i: (0, 0))


def _vec_spec():
    return pl.BlockSpec((1, _F), lambda i: (0, 0))


def _dense_layer(h, a0, a1, W, b2, p2):
    return pl.pallas_call(
        _dense_body,
        grid=(_N // _BR,),
        in_specs=[_row_spec(), _row_spec(), _row_spec(),
                  _full_spec(), _vec_spec(), _vec_spec()],
        out_specs=_row_spec(),
        out_shape=jax.ShapeDtypeStruct((_N, _F), jnp.float32),
    )(h, a0, a1, W, b2, p2)


def _dense_final_layer(h, a0, a1, W, b2, p2, Wl, bl2):
    return pl.pallas_call(
        _dense_final_body,
        grid=(_N // _BR,),
        in_specs=[_row_spec(), _row_spec(), _row_spec(),
                  _full_spec(), _vec_spec(), _vec_spec(),
                  _full_spec(), _vec_spec()],
        out_specs=_row_spec(),
        out_shape=jax.ShapeDtypeStruct((_N, _F), jnp.float32),
    )(h, a0, a1, W, b2, p2, Wl, bl2)


def kernel(x, edge_index, W0, b0, W1, b1, W2, b2, p, Wlin, blin):
    src2 = edge_index[0].reshape(_NW, _CPW, _C)
    dst2 = edge_index[1].reshape(_NW, _CPW, _C)
    zeros = jnp.zeros((_N, _F), jnp.float32)
    p2 = p.reshape(1, _F)
    b0r = b0.reshape(1, _F)
    b1r = b1.reshape(1, _F)
    b2r = b2.reshape(1, _F)
    Wlp = jnp.zeros((_F, _F), jnp.float32).at[:, :_NCLS].set(Wlin)
    blp = jnp.zeros((1, _F), jnp.float32).at[0, :_NCLS].set(blin)

    h = x
    for (W, br) in ((W0, b0r), (W1, b1r)):
        part = _sc_aggregate(h, src2, dst2, zeros)
        h = _dense_layer(h, part[:_N], part[_N:], W, br, p2)
    part = _sc_aggregate(h, src2, dst2, zeros)
    out = _dense_final_layer(h, part[:_N], part[_N:], W2, b2r, p2, Wlp, blp)
    return out[:, :_NCLS]


# speedup vs baseline: 11.6895x; 1.6854x over previous
import functools

import jax
import jax.numpy as jnp
from jax import lax
from jax.experimental import pallas as pl
from jax.experimental.pallas import tpu as pltpu
from jax.experimental.pallas import tpu_sc as plsc

_N = 10000
_E = 320000
_F = 128
_NCLS = 40
_NC = 2
_NS = 16
_NW = _NC * _NS
_C = 125
_EPW = _E // _NW
_CPW = _EPW // _C
_IB = 16
_NBLK = _CPW // _IB
_NZ = 10
_RPS = _N // _NZ


def _sc_agg_body(h_hbm, src_hbm, dst_hbm, zero_hbm, out_hbm,
                 src_v, dst_v, rows_v, agg_sh, gsem0, gsem1, isem):
    cid = lax.axis_index("c")
    sid = lax.axis_index("s")
    w = sid * _NC + cid
    gsems = (gsem0, gsem1)

    @pl.when(sid < _NZ)
    def _zero():
        pltpu.sync_copy(zero_hbm.at[pl.ds(sid * _RPS, _RPS)],
                        agg_sh.at[pl.ds(sid * _RPS, _RPS)])
    plsc.subcore_barrier()

    pltpu.sync_copy(src_hbm.at[w, pl.ds(0, _IB)], src_v.at[0])
    pltpu.sync_copy(dst_hbm.at[w, pl.ds(0, _IB)], dst_v.at[0])
    pltpu.async_copy(h_hbm.at[src_v.at[0, 0]], rows_v.at[0], gsems[0])
    pltpu.async_copy(h_hbm.at[src_v.at[0, 1]], rows_v.at[1], gsems[1])

    for k in range(_NBLK):
        kb = k % 2
        kn = (k + 1) % 2
        last = k + 1 == _NBLK
        if not last:
            pltpu.async_copy(src_hbm.at[w, pl.ds((k + 1) * _IB, _IB)],
                             src_v.at[kn], isem)
            pltpu.async_copy(dst_hbm.at[w, pl.ds((k + 1) * _IB, _IB)],
                             dst_v.at[kn], isem)

        def pair(i, carry, kb=kb):
            for b in range(2):
                j = 2 * i + b
                pltpu.make_async_copy(h_hbm.at[src_v.at[kb, j]],
                                      rows_v.at[b], gsems[b]).wait()
                pltpu.sync_copy(rows_v.at[b], agg_sh.at[dst_v.at[kb, j]],
                                add=True)
                pltpu.async_copy(h_hbm.at[src_v.at[kb, j + 2]],
                                 rows_v.at[b], gsems[b])
            return carry

        lax.fori_loop(0, (_IB - 2) // 2, pair, 0)

        if not last:
            pltpu.make_async_copy(src_hbm.at[w, pl.ds((k + 1) * _IB, _IB)],
                                  src_v.at[kn], isem).wait()
            pltpu.make_async_copy(dst_hbm.at[w, pl.ds((k + 1) * _IB, _IB)],
                                  dst_v.at[kn], isem).wait()
        for b in range(2):
            j = _IB - 2 + b
            pltpu.make_async_copy(h_hbm.at[src_v.at[kb, j]],
                                  rows_v.at[b], gsems[b]).wait()
            pltpu.sync_copy(rows_v.at[b], agg_sh.at[dst_v.at[kb, j]],
                            add=True)
            if not last:
                pltpu.async_copy(h_hbm.at[src_v.at[kn, b]],
                                 rows_v.at[b], gsems[b])

    plsc.subcore_barrier()

    @pl.when(sid < _NZ)
    def _writeout():
        pltpu.sync_copy(agg_sh.at[pl.ds(sid * _RPS, _RPS)],
                        out_hbm.at[pl.ds(cid * _N + sid * _RPS, _RPS)])


_sc_aggregate = pl.kernel(
    _sc_agg_body,
    out_type=jax.ShapeDtypeStruct((2 * _N, _F), jnp.float32),
    mesh=plsc.VectorSubcoreMesh(core_axis_name="c", subcore_axis_name="s"),
    scratch_types=[
        pltpu.VMEM((2, _IB, _C), jnp.int32),
        pltpu.VMEM((2, _IB, _C), jnp.int32),
        pltpu.VMEM((2, _C, _F), jnp.float32),
        pltpu.VMEM_SHARED((_N, _F), jnp.float32),
    ] + [pltpu.SemaphoreType.DMA] * 3,
)


def _dense_body(x_ref, a0_ref, a1_ref, w_ref, b_ref, p_ref, o_ref):
    s = x_ref[...] + a0_ref[...] + a1_ref[...]
    y = jnp.dot(s, w_ref[...], preferred_element_type=jnp.float32) + b_ref[...]
    y = jnp.maximum(y, 0.0)
    o_ref[...] = y * jnp.clip(p_ref[...], 0.0, 1.0)


def _dense_final_body(x_ref, a0_ref, a1_ref, w_ref, b_ref, p_ref,
                      wl_ref, bl_ref, o_ref):
    s = x_ref[...] + a0_ref[...] + a1_ref[...]
    y = jnp.dot(s, w_ref[...], preferred_element_type=jnp.float32) + b_ref[...]
    y = jnp.maximum(y, 0.0)
    y = y * jnp.clip(p_ref[...], 0.0, 1.0)
    o_ref[...] = jnp.dot(y, wl_ref[...], preferred_element_type=jnp.float32) + bl_ref[...]


_BR = 1000


def _row_spec():
    return pl.BlockSpec((_BR, _F), lambda i: (i, 0))


def _full_spec():
    return pl.BlockSpec((_F, _F), lambda i: (0, 0))


def _vec_spec():
    return pl.BlockSpec((1, _F), lambda i: (0, 0))


def _dense_layer(h, a0, a1, W, b2, p2):
    return pl.pallas_call(
        _dense_body,
        grid=(_N // _BR,),
        in_specs=[_row_spec(), _row_spec(), _row_spec(),
                  _full_spec(), _vec_spec(), _vec_spec()],
        out_specs=_row_spec(),
        out_shape=jax.ShapeDtypeStruct((_N, _F), jnp.float32),
    )(h, a0, a1, W, b2, p2)


def _dense_final_layer(h, a0, a1, W, b2, p2, Wl, bl2):
    return pl.pallas_call(
        _dense_final_body,
        grid=(_N // _BR,),
        in_specs=[_row_spec(), _row_spec(), _row_spec(),
                  _full_spec(), _vec_spec(), _vec_spec(),
                  _full_spec(), _vec_spec()],
        out_specs=_row_spec(),
        out_shape=jax.ShapeDtypeStruct((_N, _F), jnp.float32),
    )(h, a0, a1, W, b2, p2, Wl, bl2)


def kernel(x, edge_index, W0, b0, W1, b1, W2, b2, p, Wlin, blin):
    src2 = edge_index[0].reshape(_NW, _CPW, _C)
    dst2 = edge_index[1].reshape(_NW, _CPW, _C)
    zeros = jnp.zeros((_N, _F), jnp.float32)
    p2 = p.reshape(1, _F)
    b0r = b0.reshape(1, _F)
    b1r = b1.reshape(1, _F)
    b2r = b2.reshape(1, _F)
    Wlp = jnp.zeros((_F, _F), jnp.float32).at[:, :_NCLS].set(Wlin)
    blp = jnp.zeros((1, _F), jnp.float32).at[0, :_NCLS].set(blin)

    h = x
    for (W, br) in ((W0, b0r), (W1, b1r)):
        part = _sc_aggregate(h, src2, dst2, zeros)
        h = _dense_layer(h, part[:_N], part[_N:], W, br, p2)
    part = _sc_aggregate(h, src2, dst2, zeros)
    out = _dense_final_layer(h, part[:_N], part[_N:], W2, b2r, p2, Wlp, blp)
    return out[:, :_NCLS]


# speedup vs baseline: 12.3788x; 1.0590x over previous
import functools

import jax
import jax.numpy as jnp
from jax import lax
from jax.experimental import pallas as pl
from jax.experimental.pallas import tpu as pltpu
from jax.experimental.pallas import tpu_sc as plsc

_N = 10000
_E = 320000
_F = 128
_NCLS = 40
_NC = 2
_NS = 16
_NW = _NC * _NS
_C = 125
_EPW = _E // _NW
_CPW = _EPW // _C
_IB = 16
_NBLK = _CPW // _IB
_NZ = 10
_RPS = _N // _NZ


def _sc_agg_body(h_hbm, src_hbm, dst_hbm, zero_hbm, out_hbm,
                 src_v, dst_v, rows_v, agg_sh, gsem0, gsem1, isem):
    cid = lax.axis_index("c")
    sid = lax.axis_index("s")
    w = sid * _NC + cid
    gsems = (gsem0, gsem1)

    @pl.when(sid < _NZ)
    def _zero():
        pltpu.sync_copy(zero_hbm.at[pl.ds(sid * _RPS, _RPS)],
                        agg_sh.at[pl.ds(sid * _RPS, _RPS)])
    plsc.subcore_barrier()

    pltpu.sync_copy(src_hbm.at[w, pl.ds(0, _IB)], src_v.at[0])
    pltpu.sync_copy(dst_hbm.at[w, pl.ds(0, _IB)], dst_v.at[0])
    pltpu.async_copy(h_hbm.at[src_v.at[0, 0]], rows_v.at[0], gsems[0])
    pltpu.async_copy(h_hbm.at[src_v.at[0, 1]], rows_v.at[1], gsems[1])

    for k in range(_NBLK):
        kb = k % 2
        kn = (k + 1) % 2
        last = k + 1 == _NBLK
        if not last:
            pltpu.async_copy(src_hbm.at[w, pl.ds((k + 1) * _IB, _IB)],
                             src_v.at[kn], isem)
            pltpu.async_copy(dst_hbm.at[w, pl.ds((k + 1) * _IB, _IB)],
                             dst_v.at[kn], isem)

        def pair(i, carry, kb=kb):
            for b in range(2):
                j = 2 * i + b
                pltpu.make_async_copy(h_hbm.at[src_v.at[kb, j]],
                                      rows_v.at[b], gsems[b]).wait()
                pltpu.sync_copy(rows_v.at[b], agg_sh.at[dst_v.at[kb, j]],
                                add=True)
                pltpu.async_copy(h_hbm.at[src_v.at[kb, j + 2]],
                                 rows_v.at[b], gsems[b])
            return carry

        lax.fori_loop(0, (_IB - 2) // 2, pair, 0)

        if not last:
            pltpu.make_async_copy(src_hbm.at[w, pl.ds((k + 1) * _IB, _IB)],
                                  src_v.at[kn], isem).wait()
            pltpu.make_async_copy(dst_hbm.at[w, pl.ds((k + 1) * _IB, _IB)],
                                  dst_v.at[kn], isem).wait()
        for b in range(2):
            j = _IB - 2 + b
            pltpu.make_async_copy(h_hbm.at[src_v.at[kb, j]],
                                  rows_v.at[b], gsems[b]).wait()
            pltpu.sync_copy(rows_v.at[b], agg_sh.at[dst_v.at[kb, j]],
                            add=True)
            if not last:
                pltpu.async_copy(h_hbm.at[src_v.at[kn, b]],
                                 rows_v.at[b], gsems[b])

    plsc.subcore_barrier()

    @pl.when(sid < _NZ)
    def _writeout():
        pltpu.sync_copy(agg_sh.at[pl.ds(sid * _RPS, _RPS)],
                        out_hbm.at[pl.ds(cid * _N + sid * _RPS, _RPS)])


_sc_aggregate = pl.kernel(
    _sc_agg_body,
    out_type=jax.ShapeDtypeStruct((2 * _N, _F), jnp.float32),
    mesh=plsc.VectorSubcoreMesh(core_axis_name="c", subcore_axis_name="s"),
    scratch_types=[
        pltpu.VMEM((2, _IB, _C), jnp.int32),
        pltpu.VMEM((2, _IB, _C), jnp.int32),
        pltpu.VMEM((2, _C, _F), jnp.float32),
        pltpu.VMEM_SHARED((_N, _F), jnp.float32),
    ] + [pltpu.SemaphoreType.DMA] * 3,
)


def _dense_body(x_ref, a0_ref, a1_ref, w_ref, b_ref, p_ref, o_ref):
    s = x_ref[...] + a0_ref[...] + a1_ref[...]
    y = jnp.dot(s, w_ref[...], preferred_element_type=jnp.float32) + b_ref[...]
    y = jnp.maximum(y, 0.0)
    o_ref[...] = y * jnp.clip(p_ref[...], 0.0, 1.0)


def _dense_final_body(x_ref, a0_ref, a1_ref, w_ref, b_ref, p_ref,
                      wl_ref, bl_ref, o_ref):
    s = x_ref[...] + a0_ref[...] + a1_ref[...]
    y = jnp.dot(s, w_ref[...], preferred_element_type=jnp.float32) + b_ref[...]
    y = jnp.maximum(y, 0.0)
    y = y * jnp.clip(p_ref[...], 0.0, 1.0)
    o_ref[...] = jnp.dot(y, wl_ref[...], preferred_element_type=jnp.float32) + bl_ref[...]


_BR = 1000


def _row_spec():
    return pl.BlockSpec((_BR, _F), lambda i: (i, 0))


def _row_spec_hi():
    return pl.BlockSpec((_BR, _F), lambda i: (i + _N // _BR, 0))


def _full_spec():
    return pl.BlockSpec((_F, _F), lambda i: (0, 0))


def _vec_spec():
    return pl.BlockSpec((1, _F), lambda i: (0, 0))


def _dense_layer(h, part, W, b2, p2):
    return pl.pallas_call(
        _dense_body,
        grid=(_N // _BR,),
        in_specs=[_row_spec(), _row_spec(), _row_spec_hi(),
                  _full_spec(), _vec_spec(), _vec_spec()],
        out_specs=_row_spec(),
        out_shape=jax.ShapeDtypeStruct((_N, _F), jnp.float32),
    )(h, part, part, W, b2, p2)


def _dense_final_layer(h, part, W, b2, p2, Wl, bl2):
    return pl.pallas_call(
        _dense_final_body,
        grid=(_N // _BR,),
        in_specs=[_row_spec(), _row_spec(), _row_spec_hi(),
                  _full_spec(), _vec_spec(), _vec_spec(),
                  _full_spec(), _vec_spec()],
        out_specs=_row_spec(),
        out_shape=jax.ShapeDtypeStruct((_N, _F), jnp.float32),
    )(h, part, part, W, b2, p2, Wl, bl2)


def kernel(x, edge_index, W0, b0, W1, b1, W2, b2, p, Wlin, blin):
    src2 = edge_index[0].reshape(_NW, _CPW, _C)
    dst2 = edge_index[1].reshape(_NW, _CPW, _C)
    zeros = jnp.zeros((_N, _F), jnp.float32)
    p2 = p.reshape(1, _F)
    b0r = b0.reshape(1, _F)
    b1r = b1.reshape(1, _F)
    b2r = b2.reshape(1, _F)
    Wlp = jnp.zeros((_F, _F), jnp.float32).at[:, :_NCLS].set(Wlin)
    blp = jnp.zeros((1, _F), jnp.float32).at[0, :_NCLS].set(blin)

    h = x
    for (W, br) in ((W0, b0r), (W1, b1r)):
        part = _sc_aggregate(h, src2, dst2, zeros)
        h = _dense_layer(h, part, W, br, p2)
    part = _sc_aggregate(h, src2, dst2, zeros)
    out = _dense_final_layer(h, part, W2, b2r, p2, Wlp, blp)
    return out[:, :_NCLS]
